# initial kernel scaffold (unmeasured)
import jax
import jax.numpy as jnp
from jax import lax
from jax.experimental import pallas as pl
from jax.experimental.pallas import tpu as pltpu


def kernel(
    x,
):
    def body(*refs):
        pass

    out_shape = jax.ShapeDtypeStruct(..., jnp.float32)
    return pl.pallas_call(body, out_shape=out_shape)(...)


# baseline (device time: 307129 ns/iter reference)
import jax
import jax.numpy as jnp
from jax import lax
from jax.experimental import pallas as pl
from jax.experimental.pallas import tpu as pltpu

N_DEV = 4


def kernel(x):
    m, n = x.shape
    chunk = m // N_DEV

    def body(x_hbm, out_ref, xs, rs_send, rs_recv,
             copy_sems, send_sems, rs_sems, ag_sems):
        my = lax.axis_index("i")
        left = (my + N_DEV - 1) % N_DEV
        right = (my + 1) % N_DEV

        def rows(k):
            return pl.ds(k * chunk, chunk)

        cp = pltpu.make_async_copy(x_hbm.at[rows(my)], xs.at[0], copy_sems.at[0])
        cp.start()

        barrier_sem = pltpu.get_barrier_semaphore()
        for nbr in (left, right):
            pl.semaphore_signal(
                barrier_sem, inc=1,
                device_id=(nbr,), device_id_type=pl.DeviceIdType.MESH,
            )
        pl.semaphore_wait(barrier_sem, 2)

        cp.wait()
        rs_send[0, :, :] = xs[0, :, :].astype(jnp.bfloat16)

        for s in range(N_DEV - 1):
            k = (my + N_DEV - 1 - s) % N_DEV
            slot = (s + 1) % 2
            cp = pltpu.make_async_copy(
                x_hbm.at[rows(k)], xs.at[slot], copy_sems.at[slot]
            )
            cp.start()

            rdma = pltpu.make_async_remote_copy(
                src_ref=rs_send.at[s],
                dst_ref=rs_recv.at[s],
                send_sem=send_sems.at[s],
                recv_sem=rs_sems.at[s],
                device_id=(right,),
                device_id_type=pl.DeviceIdType.MESH,
            )
            rdma.start()
            rdma.wait()
            cp.wait()

            if s < N_DEV - 2:
                rs_send[s + 1, :, :] = (
                    rs_recv[s, :, :] + xs[slot, :, :].astype(jnp.bfloat16)
                )
            else:
                out_ref[rows(k), :] = (
                    rs_recv[s, :, :] + xs[slot, :, :].astype(jnp.bfloat16)
                )

        for t in range(N_DEV - 1):
            a = (my + 1 + N_DEV - t) % N_DEV
            rdma = pltpu.make_async_remote_copy(
                src_ref=out_ref.at[rows(a), :],
                dst_ref=out_ref.at[rows(a), :],
                send_sem=send_sems.at[N_DEV - 1 + t],
                recv_sem=ag_sems.at[t],
                device_id=(right,),
                device_id_type=pl.DeviceIdType.MESH,
            )
            rdma.start()
            rdma.wait()

    return pl.pallas_call(
        body,
        out_shape=jax.ShapeDtypeStruct((m, n), jnp.bfloat16),
        in_specs=[pl.BlockSpec(memory_space=pl.ANY)],
        out_specs=pl.BlockSpec(memory_space=pltpu.VMEM),
        scratch_shapes=[
            pltpu.VMEM((2, chunk, n), jnp.float32),
            pltpu.VMEM((N_DEV - 1, chunk, n), jnp.bfloat16),
            pltpu.VMEM((N_DEV - 1, chunk, n), jnp.bfloat16),
            pltpu.SemaphoreType.DMA((2,)),
            pltpu.SemaphoreType.DMA((2 * (N_DEV - 1),)),
            pltpu.SemaphoreType.DMA((N_DEV - 1,)),
            pltpu.SemaphoreType.DMA((N_DEV - 1,)),
        ],
        compiler_params=pltpu.CompilerParams(
            collective_id=0,
            vmem_limit_bytes=64 * 1024 * 1024,
        ),
    )(x)


# device time: 173977 ns/iter; 1.7653x vs baseline; 1.7653x over previous
import jax
import jax.numpy as jnp
from jax import lax
from jax.experimental import pallas as pl
from jax.experimental.pallas import tpu as pltpu

N_DEV = 4
CW, CCW = 0, 1


def kernel(x):
    m, n = x.shape
    half = m // 2
    chunk = half // N_DEV

    def body(x_hbm, out_ref, xs, rs_send, rs_recv,
             copy_sems, send_sems, rs_sems, ag_sems):
        my = lax.axis_index("i")
        left = (my + N_DEV - 1) % N_DEV
        right = (my + 1) % N_DEV

        def rows(d, k):
            return pl.ds(d * half + k * chunk, chunk)

        def stage(d, k, slot):
            cp = pltpu.make_async_copy(
                x_hbm.at[rows(d, k)], xs.at[d, slot], copy_sems.at[d, slot]
            )
            cp.start()
            return cp

        cps = [stage(d, my, 0) for d in (CW, CCW)]

        barrier_sem = pltpu.get_barrier_semaphore()
        for nbr in (left, right):
            pl.semaphore_signal(
                barrier_sem, inc=1,
                device_id=(nbr,), device_id_type=pl.DeviceIdType.MESH,
            )
        pl.semaphore_wait(barrier_sem, 2)

        for cp in cps:
            cp.wait()
        rs_send[CW, 0, :, :] = xs[CW, 0, :, :].astype(jnp.bfloat16)
        rs_send[CCW, 0, :, :] = xs[CCW, 0, :, :].astype(jnp.bfloat16)

        for s in range(N_DEV - 1):
            k_cw = (my + N_DEV - 1 - s) % N_DEV
            k_ccw = (my + 1 + s) % N_DEV
            slot = (s + 1) % 2
            cps = [stage(CW, k_cw, slot), stage(CCW, k_ccw, slot)]

            rdmas = []
            for d, dst in ((CW, right), (CCW, left)):
                rdma = pltpu.make_async_remote_copy(
                    src_ref=rs_send.at[d, s],
                    dst_ref=rs_recv.at[d, s],
                    send_sem=send_sems.at[d, s],
                    recv_sem=rs_sems.at[d, s],
                    device_id=(dst,),
                    device_id_type=pl.DeviceIdType.MESH,
                )
                rdma.start()
                rdmas.append(rdma)
            for rdma in rdmas:
                rdma.wait()
            for cp in cps:
                cp.wait()

            for d, k in ((CW, k_cw), (CCW, k_ccw)):
                if s < N_DEV - 2:
                    rs_send[d, s + 1, :, :] = (
                        rs_recv[d, s, :, :] + xs[d, slot, :, :].astype(jnp.bfloat16)
                    )
                else:
                    out_ref[rows(d, k), :] = (
                        rs_recv[d, s, :, :] + xs[d, slot, :, :].astype(jnp.bfloat16)
                    )

        for t in range(N_DEV - 1):
            a_cw = (my + 1 + N_DEV - t) % N_DEV
            a_ccw = (my + N_DEV - 1 + t) % N_DEV
            rdmas = []
            for d, a, dst in ((CW, a_cw, right), (CCW, a_ccw, left)):
                rdma = pltpu.make_async_remote_copy(
                    src_ref=out_ref.at[rows(d, a), :],
                    dst_ref=out_ref.at[rows(d, a), :],
                    send_sem=send_sems.at[d, N_DEV - 1 + t],
                    recv_sem=ag_sems.at[d, t],
                    device_id=(dst,),
                    device_id_type=pl.DeviceIdType.MESH,
                )
                rdma.start()
                rdmas.append(rdma)
            for rdma in rdmas:
                rdma.wait()

    return pl.pallas_call(
        body,
        out_shape=jax.ShapeDtypeStruct((m, n), jnp.bfloat16),
        in_specs=[pl.BlockSpec(memory_space=pl.ANY)],
        out_specs=pl.BlockSpec(memory_space=pltpu.VMEM),
        scratch_shapes=[
            pltpu.VMEM((2, 2, chunk, n), jnp.float32),
            pltpu.VMEM((2, N_DEV - 1, chunk, n), jnp.bfloat16),
            pltpu.VMEM((2, N_DEV - 1, chunk, n), jnp.bfloat16),
            pltpu.SemaphoreType.DMA((2, 2)),
            pltpu.SemaphoreType.DMA((2, 2 * (N_DEV - 1))),
            pltpu.SemaphoreType.DMA((2, N_DEV - 1)),
            pltpu.SemaphoreType.DMA((2, N_DEV - 1)),
        ],
        compiler_params=pltpu.CompilerParams(
            collective_id=0,
            vmem_limit_bytes=64 * 1024 * 1024,
        ),
    )(x)


# device time: 171017 ns/iter; 1.7959x vs baseline; 1.0173x over previous
import jax
import jax.numpy as jnp
from jax import lax
from jax.experimental import pallas as pl
from jax.experimental.pallas import tpu as pltpu

N_DEV = 4
CW, CCW = 0, 1


def kernel(x):
    m, n = x.shape
    half = m // 2
    chunk = half // N_DEV

    def body(x_hbm, out_hbm, xs, rs_send, rs_recv, red,
             copy_sems, out_sems, send_sems, rs_sems, ag_sems):
        my = lax.axis_index("i")
        left = (my + N_DEV - 1) % N_DEV
        right = (my + 1) % N_DEV

        def rows(d, k):
            return pl.ds(d * half + k * chunk, chunk)

        def stage(d, k, slot):
            cp = pltpu.make_async_copy(
                x_hbm.at[rows(d, k)], xs.at[d, slot], copy_sems.at[d, slot]
            )
            cp.start()
            return cp

        cps = [stage(d, my, 0) for d in (CW, CCW)]

        barrier_sem = pltpu.get_barrier_semaphore()
        for nbr in (left, right):
            pl.semaphore_signal(
                barrier_sem, inc=1,
                device_id=(nbr,), device_id_type=pl.DeviceIdType.MESH,
            )
        pl.semaphore_wait(barrier_sem, 2)

        for cp in cps:
            cp.wait()
        rs_send[CW, 0, :, :] = xs[CW, 0, :, :].astype(jnp.bfloat16)
        rs_send[CCW, 0, :, :] = xs[CCW, 0, :, :].astype(jnp.bfloat16)

        for s in range(N_DEV - 1):
            k_cw = (my + N_DEV - 1 - s) % N_DEV
            k_ccw = (my + 1 + s) % N_DEV
            slot = (s + 1) % 2
            cps = [stage(CW, k_cw, slot), stage(CCW, k_ccw, slot)]

            rdmas = []
            for d, dst in ((CW, right), (CCW, left)):
                rdma = pltpu.make_async_remote_copy(
                    src_ref=rs_send.at[d, s],
                    dst_ref=rs_recv.at[d, s],
                    send_sem=send_sems.at[d, s],
                    recv_sem=rs_sems.at[d, s],
                    device_id=(dst,),
                    device_id_type=pl.DeviceIdType.MESH,
                )
                rdma.start()
                rdmas.append(rdma)
            for rdma in rdmas:
                rdma.wait()
            for cp in cps:
                cp.wait()

            out_cps = []
            for d, k in ((CW, k_cw), (CCW, k_ccw)):
                if s < N_DEV - 2:
                    rs_send[d, s + 1, :, :] = (
                        rs_recv[d, s, :, :] + xs[d, slot, :, :].astype(jnp.bfloat16)
                    )
                else:
                    red[d, :, :] = (
                        rs_recv[d, s, :, :] + xs[d, slot, :, :].astype(jnp.bfloat16)
                    )
                    cp = pltpu.make_async_copy(
                        red.at[d], out_hbm.at[rows(d, k)], out_sems.at[d]
                    )
                    cp.start()
                    out_cps.append(cp)
            for cp in out_cps:
                cp.wait()

        for t in range(N_DEV - 1):
            a_cw = (my + 1 + N_DEV - t) % N_DEV
            a_ccw = (my + N_DEV - 1 + t) % N_DEV
            rdmas = []
            for d, a, dst in ((CW, a_cw, right), (CCW, a_ccw, left)):
                rdma = pltpu.make_async_remote_copy(
                    src_ref=out_hbm.at[rows(d, a), :],
                    dst_ref=out_hbm.at[rows(d, a), :],
                    send_sem=send_sems.at[d, N_DEV - 1 + t],
                    recv_sem=ag_sems.at[d, t],
                    device_id=(dst,),
                    device_id_type=pl.DeviceIdType.MESH,
                )
                rdma.start()
                rdmas.append(rdma)
            for rdma in rdmas:
                rdma.wait()

    return pl.pallas_call(
        body,
        out_shape=jax.ShapeDtypeStruct((m, n), jnp.bfloat16),
        in_specs=[pl.BlockSpec(memory_space=pl.ANY)],
        out_specs=pl.BlockSpec(memory_space=pl.ANY),
        scratch_shapes=[
            pltpu.VMEM((2, 2, chunk, n), jnp.float32),
            pltpu.VMEM((2, N_DEV - 1, chunk, n), jnp.bfloat16),
            pltpu.VMEM((2, N_DEV - 1, chunk, n), jnp.bfloat16),
            pltpu.VMEM((2, chunk, n), jnp.bfloat16),
            pltpu.SemaphoreType.DMA((2, 2)),
            pltpu.SemaphoreType.DMA((2,)),
            pltpu.SemaphoreType.DMA((2, 2 * (N_DEV - 1))),
            pltpu.SemaphoreType.DMA((2, N_DEV - 1)),
            pltpu.SemaphoreType.DMA((2, N_DEV - 1)),
        ],
        compiler_params=pltpu.CompilerParams(
            collective_id=0,
            vmem_limit_bytes=64 * 1024 * 1024,
        ),
    )(x)


# device time: 156661 ns/iter; 1.9605x vs baseline; 1.0916x over previous
import jax
import jax.numpy as jnp
from jax import lax
from jax.experimental import pallas as pl
from jax.experimental.pallas import tpu as pltpu

N_DEV = 4
CW, CCW = 0, 1
SUB = 2


def kernel(x):
    m, n = x.shape
    half = m // 2
    chunk = half // N_DEV
    sr = chunk // SUB

    def body(x_hbm, out_hbm, xs, rs_send, rs_recv, red, ag_buf,
             stage_sems, out_sems, rs_ssems, rs_rsems, ag_ssems, ag_rsems):
        my = lax.axis_index("i")
        left = (my + N_DEV - 1) % N_DEV
        right = (my + 1) % N_DEV
        peer = {CW: right, CCW: left}

        def rows(d, k):
            return pl.ds(d * half + k * chunk, chunk)

        def rows_sub(d, k, j):
            return pl.ds(d * half + k * chunk + j * sr, sr)

        def sub(j):
            return pl.ds(j * sr, sr)

        def stage(d, k, slot):
            cp = pltpu.make_async_copy(
                x_hbm.at[rows(d, k)], xs.at[d, slot], stage_sems.at[d, slot]
            )
            cp.start()
            return cp

        def rs_desc(d, s, j):
            return pltpu.make_async_remote_copy(
                src_ref=rs_send.at[d, s, sub(j)],
                dst_ref=rs_recv.at[d, s, sub(j)],
                send_sem=rs_ssems.at[d, s, j],
                recv_sem=rs_rsems.at[d, s, j],
                device_id=(peer[d],),
                device_id_type=pl.DeviceIdType.MESH,
            )

        def ag_desc(d, t, j):
            src = red.at[d, sub(j)] if t == 0 else ag_buf.at[d, t - 1, sub(j)]
            return pltpu.make_async_remote_copy(
                src_ref=src,
                dst_ref=ag_buf.at[d, t, sub(j)],
                send_sem=ag_ssems.at[d, t, j],
                recv_sem=ag_rsems.at[d, t, j],
                device_id=(peer[d],),
                device_id_type=pl.DeviceIdType.MESH,
            )

        init_cps = {
            (d, j): pltpu.make_async_copy(
                x_hbm.at[rows_sub(d, my, j)],
                xs.at[d, 0, sub(j)],
                stage_sems.at[d, j],
            )
            for d in (CW, CCW)
            for j in range(SUB)
        }
        for cp in init_cps.values():
            cp.start()

        barrier_sem = pltpu.get_barrier_semaphore()
        for nbr in (left, right):
            pl.semaphore_signal(
                barrier_sem, inc=1,
                device_id=(nbr,), device_id_type=pl.DeviceIdType.MESH,
            )
        pl.semaphore_wait(barrier_sem, 2)

        for j in range(SUB):
            for d in (CW, CCW):
                init_cps[(d, j)].wait()
                rs_send[d, 0, j * sr:(j + 1) * sr, :] = (
                    xs[d, 0, j * sr:(j + 1) * sr, :].astype(jnp.bfloat16)
                )
                rs_desc(d, 0, j).start()

        out_cps = []
        r_idx = {CW: (my + 1) % N_DEV, CCW: (my + N_DEV - 1) % N_DEV}

        for s in range(N_DEV - 1):
            slot = (s + 1) % 2
            k = {CW: (my + N_DEV - 1 - s) % N_DEV, CCW: (my + 1 + s) % N_DEV}
            cps = [stage(d, k[d], slot) for d in (CW, CCW)]
            for j in range(SUB):
                for d in (CW, CCW):
                    rs_desc(d, s, j).wait()
                if j == 0:
                    for cp in cps:
                        cp.wait()
                rsub = slice(j * sr, (j + 1) * sr)
                for d in (CW, CCW):
                    if s < N_DEV - 2:
                        rs_send[d, s + 1, rsub, :] = (
                            rs_recv[d, s, rsub, :]
                            + xs[d, slot, rsub, :].astype(jnp.bfloat16)
                        )
                        rs_desc(d, s + 1, j).start()
                    else:
                        red[d, rsub, :] = (
                            rs_recv[d, s, rsub, :]
                            + xs[d, slot, rsub, :].astype(jnp.bfloat16)
                        )
                        ag_desc(d, 0, j).start()
                        cp = pltpu.make_async_copy(
                            red.at[d, sub(j)],
                            out_hbm.at[rows_sub(d, r_idx[d], j)],
                            out_sems.at[d, N_DEV - 1, j],
                        )
                        cp.start()
                        out_cps.append(cp)

        for t in range(N_DEV - 1):
            kr = {CW: (my + N_DEV - t) % N_DEV, CCW: (my + t) % N_DEV}
            for j in range(SUB):
                for d in (CW, CCW):
                    ag_desc(d, t, j).wait()
                for d in (CW, CCW):
                    if t < N_DEV - 2:
                        ag_desc(d, t + 1, j).start()
                    cp = pltpu.make_async_copy(
                        ag_buf.at[d, t, sub(j)],
                        out_hbm.at[rows_sub(d, kr[d], j)],
                        out_sems.at[d, t, j],
                    )
                    cp.start()
                    out_cps.append(cp)

        for cp in out_cps:
            cp.wait()

    return pl.pallas_call(
        body,
        out_shape=jax.ShapeDtypeStruct((m, n), jnp.bfloat16),
        in_specs=[pl.BlockSpec(memory_space=pl.ANY)],
        out_specs=pl.BlockSpec(memory_space=pl.ANY),
        scratch_shapes=[
            pltpu.VMEM((2, 2, chunk, n), jnp.float32),
            pltpu.VMEM((2, N_DEV - 1, chunk, n), jnp.bfloat16),
            pltpu.VMEM((2, N_DEV - 1, chunk, n), jnp.bfloat16),
            pltpu.VMEM((2, chunk, n), jnp.bfloat16),
            pltpu.VMEM((2, N_DEV - 1, chunk, n), jnp.bfloat16),
            pltpu.SemaphoreType.DMA((2, 2)),
            pltpu.SemaphoreType.DMA((2, N_DEV, SUB)),
            pltpu.SemaphoreType.DMA((2, N_DEV - 1, SUB)),
            pltpu.SemaphoreType.DMA((2, N_DEV - 1, SUB)),
            pltpu.SemaphoreType.DMA((2, N_DEV - 1, SUB)),
            pltpu.SemaphoreType.DMA((2, N_DEV - 1, SUB)),
        ],
        compiler_params=pltpu.CompilerParams(
            collective_id=0,
            vmem_limit_bytes=64 * 1024 * 1024,
        ),
    )(x)


# device time: 155957 ns/iter; 1.9693x vs baseline; 1.0045x over previous
import jax
import jax.numpy as jnp
from jax import lax
from jax.experimental import pallas as pl
from jax.experimental.pallas import tpu as pltpu

N_DEV = 4
CW, CCW = 0, 1
SUB = 4


def kernel(x):
    m, n = x.shape
    half = m // 2
    chunk = half // N_DEV
    sr = chunk // SUB

    def body(x_hbm, out_hbm, xs, rs_send, rs_recv, red, ag_buf,
             stage_sems, out_sems, rs_ssems, rs_rsems, ag_ssems, ag_rsems):
        my = lax.axis_index("i")
        left = (my + N_DEV - 1) % N_DEV
        right = (my + 1) % N_DEV
        peer = {CW: right, CCW: left}

        def rows(d, k):
            return pl.ds(d * half + k * chunk, chunk)

        def rows_sub(d, k, j):
            return pl.ds(d * half + k * chunk + j * sr, sr)

        def sub(j):
            return pl.ds(j * sr, sr)

        def stage(d, k, slot):
            cp = pltpu.make_async_copy(
                x_hbm.at[rows(d, k)], xs.at[d, slot], stage_sems.at[d, slot]
            )
            cp.start()
            return cp

        def rs_desc(d, s, j):
            return pltpu.make_async_remote_copy(
                src_ref=rs_send.at[d, s, sub(j)],
                dst_ref=rs_recv.at[d, s, sub(j)],
                send_sem=rs_ssems.at[d, s, j],
                recv_sem=rs_rsems.at[d, s, j],
                device_id=(peer[d],),
                device_id_type=pl.DeviceIdType.MESH,
            )

        def ag_desc(d, t, j):
            src = red.at[d, sub(j)] if t == 0 else ag_buf.at[d, t - 1, sub(j)]
            return pltpu.make_async_remote_copy(
                src_ref=src,
                dst_ref=ag_buf.at[d, t, sub(j)],
                send_sem=ag_ssems.at[d, t, j],
                recv_sem=ag_rsems.at[d, t, j],
                device_id=(peer[d],),
                device_id_type=pl.DeviceIdType.MESH,
            )

        init_cps = {
            (d, j): pltpu.make_async_copy(
                x_hbm.at[rows_sub(d, my, j)],
                xs.at[d, 0, sub(j)],
                stage_sems.at[d, j],
            )
            for d in (CW, CCW)
            for j in range(SUB)
        }
        for cp in init_cps.values():
            cp.start()

        barrier_sem = pltpu.get_barrier_semaphore()
        for nbr in (left, right):
            pl.semaphore_signal(
                barrier_sem, inc=1,
                device_id=(nbr,), device_id_type=pl.DeviceIdType.MESH,
            )
        pl.semaphore_wait(barrier_sem, 2)

        for j in range(SUB):
            for d in (CW, CCW):
                init_cps[(d, j)].wait()
                rs_send[d, 0, j * sr:(j + 1) * sr, :] = (
                    xs[d, 0, j * sr:(j + 1) * sr, :].astype(jnp.bfloat16)
                )
                rs_desc(d, 0, j).start()

        out_cps = []
        r_idx = {CW: (my + 1) % N_DEV, CCW: (my + N_DEV - 1) % N_DEV}

        for s in range(N_DEV - 1):
            slot = (s + 1) % 2
            k = {CW: (my + N_DEV - 1 - s) % N_DEV, CCW: (my + 1 + s) % N_DEV}
            cps = [stage(d, k[d], slot) for d in (CW, CCW)]
            for j in range(SUB):
                for d in (CW, CCW):
                    rs_desc(d, s, j).wait()
                if j == 0:
                    for cp in cps:
                        cp.wait()
                rsub = slice(j * sr, (j + 1) * sr)
                for d in (CW, CCW):
                    if s < N_DEV - 2:
                        rs_send[d, s + 1, rsub, :] = (
                            rs_recv[d, s, rsub, :]
                            + xs[d, slot, rsub, :].astype(jnp.bfloat16)
                        )
                        rs_desc(d, s + 1, j).start()
                    else:
                        red[d, rsub, :] = (
                            rs_recv[d, s, rsub, :]
                            + xs[d, slot, rsub, :].astype(jnp.bfloat16)
                        )
                        ag_desc(d, 0, j).start()
                        cp = pltpu.make_async_copy(
                            red.at[d, sub(j)],
                            out_hbm.at[rows_sub(d, r_idx[d], j)],
                            out_sems.at[d, N_DEV - 1, j],
                        )
                        cp.start()
                        out_cps.append(cp)

        for t in range(N_DEV - 1):
            kr = {CW: (my + N_DEV - t) % N_DEV, CCW: (my + t) % N_DEV}
            for j in range(SUB):
                for d in (CW, CCW):
                    ag_desc(d, t, j).wait()
                for d in (CW, CCW):
                    if t < N_DEV - 2:
                        ag_desc(d, t + 1, j).start()
                    cp = pltpu.make_async_copy(
                        ag_buf.at[d, t, sub(j)],
                        out_hbm.at[rows_sub(d, kr[d], j)],
                        out_sems.at[d, t, j],
                    )
                    cp.start()
                    out_cps.append(cp)

        for cp in out_cps:
            cp.wait()

    return pl.pallas_call(
        body,
        out_shape=jax.ShapeDtypeStruct((m, n), jnp.bfloat16),
        in_specs=[pl.BlockSpec(memory_space=pl.ANY)],
        out_specs=pl.BlockSpec(memory_space=pl.ANY),
        scratch_shapes=[
            pltpu.VMEM((2, 2, chunk, n), jnp.float32),
            pltpu.VMEM((2, N_DEV - 1, chunk, n), jnp.bfloat16),
            pltpu.VMEM((2, N_DEV - 1, chunk, n), jnp.bfloat16),
            pltpu.VMEM((2, chunk, n), jnp.bfloat16),
            pltpu.VMEM((2, N_DEV - 1, chunk, n), jnp.bfloat16),
            pltpu.SemaphoreType.DMA((2, 2)),
            pltpu.SemaphoreType.DMA((2, N_DEV, SUB)),
            pltpu.SemaphoreType.DMA((2, N_DEV - 1, SUB)),
            pltpu.SemaphoreType.DMA((2, N_DEV - 1, SUB)),
            pltpu.SemaphoreType.DMA((2, N_DEV - 1, SUB)),
            pltpu.SemaphoreType.DMA((2, N_DEV - 1, SUB)),
        ],
        compiler_params=pltpu.CompilerParams(
            collective_id=0,
            vmem_limit_bytes=64 * 1024 * 1024,
        ),
    )(x)
